# baseline (speedup 1.0000x reference)
import functools

import jax
import jax.numpy as jnp
from jax import lax
from jax.experimental import pallas as pl
from jax.experimental.pallas import tpu as pltpu
from jax.experimental.pallas import tpu_sc as plsc

B, L, VOCAB, DIM = 4096, 200, 8, 4
N = B * L
LANES = 128
UNITS = L * (B // LANES)


def _make_sc_call(num_cores, num_subcores):
  NW = num_cores * num_subcores
  UPW = UNITS // NW
  UB = 20
  NBLK = UPW // UB
  NSUP = NBLK // 2
  IDX_BLK = UB * LANES
  OUT_BLK = UB * LANES * VOCAB

  mesh = plsc.VectorSubcoreMesh(
      core_axis_name="c", subcore_axis_name="s",
      num_cores=num_cores, num_subcores=num_subcores)

  @functools.partial(
      pl.kernel,
      out_type=jax.ShapeDtypeStruct((N * VOCAB,), jnp.float32),
      mesh=mesh,
      scratch_types=[
          pltpu.VMEM((VOCAB * VOCAB, 16), jnp.float32),
          pltpu.VMEM((IDX_BLK,), jnp.int32),
          pltpu.VMEM((IDX_BLK,), jnp.int32),
          pltpu.VMEM((OUT_BLK,), jnp.float32),
          pltpu.VMEM((OUT_BLK,), jnp.float32),
          pltpu.SemaphoreType.DMA,
          pltpu.SemaphoreType.DMA,
          pltpu.SemaphoreType.DMA,
          pltpu.SemaphoreType.DMA,
      ],
      compiler_params=pltpu.CompilerParams(needs_layout_passes=False),
  )
  def call(idx_hbm, w_hbm, out_hbm, g_rep, idx_v0, idx_v1, out_v0, out_v1,
           sin0, sin1, sout0, sout1):
    idx_bufs = (idx_v0, idx_v1)
    out_bufs = (out_v0, out_v1)
    sins = (sin0, sin1)
    souts = (sout0, sout1)

    wid = lax.axis_index("s") * num_cores + lax.axis_index("c")
    lane = lax.iota(jnp.int32, 16)
    half = lane // 8
    col8 = lane & 7

    def build_gram(w_v):
      pltpu.sync_copy(w_hbm, w_v)
      for q in range(4):
        evec = 16 * q + lane
        ivec = 2 * q + half
        acc = jnp.zeros((16,), jnp.float32)
        for d in range(DIM):
          dvec = jnp.full((16,), d, jnp.int32)
          a = plsc.load_gather(w_v, [ivec, dvec])
          b = plsc.load_gather(w_v, [col8, dvec])
          acc = acc + a * b
        for rep in range(16):
          plsc.store_scatter(g_rep, [evec, jnp.full((16,), rep, jnp.int32)],
                             acc)

    pl.run_scoped(build_gram, pltpu.VMEM((VOCAB, DIM), jnp.float32))

    tile_tok0 = wid * (UPW * LANES)

    def in_src(blk):
      return idx_hbm.at[pl.ds(tile_tok0 + blk * IDX_BLK, IDX_BLK)]

    def out_dst(blk):
      return out_hbm.at[pl.ds((tile_tok0 + blk * IDX_BLK) * VOCAB, OUT_BLK)]

    for b in range(2):
      pltpu.async_copy(in_src(b), idx_bufs[b], sins[b])

    def super_body(s, _):
      for b in range(2):
        blk = s * 2 + b
        idx_v = idx_bufs[b]
        out_v = out_bufs[b]
        pltpu.make_async_copy(in_src(0), idx_v, sins[b]).wait()

        @pl.when(s >= 1)
        def _wait_out():
          pltpu.make_async_copy(out_v, out_dst(0), souts[b]).wait()

        @plsc.parallel_loop(0, UB, unroll=4)
        def _unit(u):
          for g in range(LANES // 16):
            vidx = idx_v[pl.ds(u * LANES + g * 16, 16)]
            e8 = vidx * VOCAB
            for v in range(VOCAB):
              vals = plsc.load_gather(g_rep, [e8 + v, lane])
              out_v[pl.ds(u * (LANES * VOCAB) + v * LANES + g * 16, 16)] = vals

        pltpu.async_copy(out_v, out_dst(blk), souts[b])

        @pl.when(s <= NSUP - 2)
        def _next_in():
          pltpu.async_copy(in_src(blk + 2), idx_v, sins[b])

      return 0

    lax.fori_loop(0, NSUP, super_body, 0)
    for b in range(2):
      pltpu.make_async_copy(out_bufs[b], out_dst(0), souts[b]).wait()

  return call


def kernel(indices, W):
  idx_t = jnp.swapaxes(indices.astype(jnp.int32), 0, 1).reshape(-1)
  info = plsc.get_sparse_core_info()
  call = _make_sc_call(info.num_cores, info.num_subcores)
  flat = call(idx_t, W)
  return (flat.reshape(L, B // LANES, VOCAB, LANES)
          .transpose(1, 3, 0, 2)
          .reshape(B, L, VOCAB))


# speedup vs baseline: 1.0642x; 1.0642x over previous
import functools

import jax
import jax.numpy as jnp
from jax import lax
from jax.experimental import pallas as pl
from jax.experimental.pallas import tpu as pltpu
from jax.experimental.pallas import tpu_sc as plsc

B, L, VOCAB, DIM = 4096, 200, 8, 4
N = B * L
LANES = 128
UNITS = L * (B // LANES)


def _make_sc_call(num_cores, num_subcores):
  NW = num_cores * num_subcores
  UPW = UNITS // NW
  UB = 25
  NBLK = UPW // UB
  NSUP = NBLK // 2
  IDX_BLK = UB * LANES
  OUT_BLK = UB * LANES * VOCAB

  mesh = plsc.VectorSubcoreMesh(
      core_axis_name="c", subcore_axis_name="s",
      num_cores=num_cores, num_subcores=num_subcores)

  @functools.partial(
      pl.kernel,
      out_type=jax.ShapeDtypeStruct((N * VOCAB,), jnp.float32),
      mesh=mesh,
      scratch_types=[
          pltpu.VMEM((VOCAB * VOCAB, 16), jnp.float32),
          pltpu.VMEM((IDX_BLK,), jnp.int32),
          pltpu.VMEM((IDX_BLK,), jnp.int32),
          pltpu.VMEM((OUT_BLK,), jnp.float32),
          pltpu.VMEM((OUT_BLK,), jnp.float32),
          pltpu.SemaphoreType.DMA,
          pltpu.SemaphoreType.DMA,
          pltpu.SemaphoreType.DMA,
          pltpu.SemaphoreType.DMA,
      ],
      compiler_params=pltpu.CompilerParams(needs_layout_passes=False),
  )
  def call(idx_hbm, w_hbm, out_hbm, g_rep, idx_v0, idx_v1, out_v0, out_v1,
           sin0, sin1, sout0, sout1):
    idx_bufs = (idx_v0, idx_v1)
    out_bufs = (out_v0, out_v1)
    sins = (sin0, sin1)
    souts = (sout0, sout1)

    wid = lax.axis_index("s") * num_cores + lax.axis_index("c")
    lane = lax.iota(jnp.int32, 16)
    half = lane // 8
    col8 = lane & 7

    def build_gram(w_v):
      pltpu.sync_copy(w_hbm, w_v)
      for q in range(4):
        evec = 16 * q + lane
        ivec = 2 * q + half
        acc = jnp.zeros((16,), jnp.float32)
        for d in range(DIM):
          dvec = jnp.full((16,), d, jnp.int32)
          a = plsc.load_gather(w_v, [ivec, dvec])
          b = plsc.load_gather(w_v, [col8, dvec])
          acc = acc + a * b
        for rep in range(16):
          plsc.store_scatter(g_rep, [evec, jnp.full((16,), rep, jnp.int32)],
                             acc)

    pl.run_scoped(build_gram, pltpu.VMEM((VOCAB, DIM), jnp.float32))

    tile_tok0 = wid * (UPW * LANES)

    def in_src(blk):
      return idx_hbm.at[pl.ds(tile_tok0 + blk * IDX_BLK, IDX_BLK)]

    def out_dst(blk):
      return out_hbm.at[pl.ds((tile_tok0 + blk * IDX_BLK) * VOCAB, OUT_BLK)]

    for b in range(2):
      pltpu.async_copy(in_src(b), idx_bufs[b], sins[b])

    def super_body(s, _):
      for b in range(2):
        blk = s * 2 + b
        idx_v = idx_bufs[b]
        out_v = out_bufs[b]
        pltpu.make_async_copy(in_src(0), idx_v, sins[b]).wait()

        @pl.when(s >= 1)
        def _wait_out():
          pltpu.make_async_copy(out_v, out_dst(0), souts[b]).wait()

        @plsc.parallel_loop(0, UB, unroll=5)
        def _unit(u):
          for g in range(LANES // 16):
            vidx = idx_v[pl.ds(u * LANES + g * 16, 16)]
            e8 = vidx * VOCAB
            for v in range(VOCAB):
              vals = plsc.load_gather(g_rep, [e8 + v, lane])
              out_v[pl.ds(u * (LANES * VOCAB) + v * LANES + g * 16, 16)] = vals

        pltpu.async_copy(out_v, out_dst(blk), souts[b])

        @pl.when(s <= NSUP - 2)
        def _next_in():
          pltpu.async_copy(in_src(blk + 2), idx_v, sins[b])

      return 0

    lax.fori_loop(0, NSUP, super_body, 0)
    for b in range(2):
      pltpu.make_async_copy(out_bufs[b], out_dst(0), souts[b]).wait()

  return call


def kernel(indices, W):
  idx_t = jnp.swapaxes(indices.astype(jnp.int32), 0, 1).reshape(-1)
  info = plsc.get_sparse_core_info()
  call = _make_sc_call(info.num_cores, info.num_subcores)
  flat = call(idx_t, W)
  return (flat.reshape(L, B // LANES, VOCAB, LANES)
          .transpose(1, 3, 0, 2)
          .reshape(B, L, VOCAB))


# speedup vs baseline: 1.3400x; 1.2592x over previous
import functools

import jax
import jax.numpy as jnp
from jax import lax
from jax.experimental import pallas as pl
from jax.experimental.pallas import tpu as pltpu
from jax.experimental.pallas import tpu_sc as plsc

B, L, VOCAB, DIM = 4096, 200, 8, 4
N = B * L
LANES = 128
UNITS = L * (B // LANES)


def _make_sc_call(num_cores, num_subcores):
  NW = num_cores * num_subcores
  UPW = UNITS // NW
  UB = 20
  NBLK = UPW // UB
  NSUP = NBLK // 2
  IDX_BLK = UB * LANES
  OUT_BLK = UB * LANES * VOCAB

  mesh = plsc.VectorSubcoreMesh(
      core_axis_name="c", subcore_axis_name="s",
      num_cores=num_cores, num_subcores=num_subcores)

  @functools.partial(
      pl.kernel,
      out_type=jax.ShapeDtypeStruct((N * VOCAB,), jnp.float32),
      mesh=mesh,
      scratch_types=[
          pltpu.VMEM((VOCAB * VOCAB, 16), jnp.float32),
          pltpu.VMEM((IDX_BLK,), jnp.int32),
          pltpu.VMEM((IDX_BLK,), jnp.int32),
          pltpu.VMEM((OUT_BLK,), jnp.float32),
          pltpu.VMEM((OUT_BLK,), jnp.float32),
          pltpu.SemaphoreType.DMA,
          pltpu.SemaphoreType.DMA,
          pltpu.SemaphoreType.DMA,
          pltpu.SemaphoreType.DMA,
      ],
      compiler_params=pltpu.CompilerParams(needs_layout_passes=False),
  )
  def call(idx_hbm, w_hbm, out_hbm, g_rep, idx_v0, idx_v1, out_v0, out_v1,
           sin0, sin1, sout0, sout1):
    idx_bufs = (idx_v0, idx_v1)
    out_bufs = (out_v0, out_v1)
    sins = (sin0, sin1)
    souts = (sout0, sout1)

    wid = lax.axis_index("s") * num_cores + lax.axis_index("c")
    lane = lax.iota(jnp.int32, 16)
    half = lane // 8
    col8 = lane & 7

    def build_gram(w_v):
      pltpu.sync_copy(w_hbm, w_v)
      for q in range(4):
        evec = 16 * q + lane
        ivec = 2 * q + half
        acc = jnp.zeros((16,), jnp.float32)
        for d in range(DIM):
          dvec = jnp.full((16,), d, jnp.int32)
          a = plsc.load_gather(w_v, [ivec, dvec])
          b = plsc.load_gather(w_v, [col8, dvec])
          acc = acc + a * b
        for rep in range(16):
          plsc.store_scatter(g_rep, [evec, jnp.full((16,), rep, jnp.int32)],
                             acc)

    pl.run_scoped(build_gram, pltpu.VMEM((VOCAB, DIM), jnp.float32))

    tile_tok0 = wid * (UPW * LANES)

    def in_src(blk):
      return idx_hbm.at[pl.ds(tile_tok0 + blk * IDX_BLK, IDX_BLK)]

    def out_dst(blk):
      return out_hbm.at[pl.ds((tile_tok0 + blk * IDX_BLK) * VOCAB, OUT_BLK)]

    for b in range(2):
      pltpu.async_copy(in_src(b), idx_bufs[b], sins[b])

    def super_body(s, _):
      for b in range(2):
        blk = s * 2 + b
        idx_v = idx_bufs[b]
        out_v = out_bufs[b]
        pltpu.make_async_copy(in_src(0), idx_v, sins[b]).wait()

        @pl.when(s >= 1)
        def _wait_out():
          pltpu.make_async_copy(out_v, out_dst(0), souts[b]).wait()

        @plsc.parallel_loop(0, UB * (LANES // 16), unroll=8)
        def _grp(t):
          u = t // 8
          g = t - u * 8
          vidx = idx_v[pl.ds(t * 16, 16)]
          e8 = vidx * VOCAB
          for v in range(VOCAB):
            vals = plsc.load_gather(g_rep, [e8 + v, lane])
            out_v[pl.ds(u * (LANES * VOCAB) + v * LANES + g * 16, 16)] = vals

        pltpu.async_copy(out_v, out_dst(blk), souts[b])

        @pl.when(s <= NSUP - 2)
        def _next_in():
          pltpu.async_copy(in_src(blk + 2), idx_v, sins[b])

      return 0

    lax.fori_loop(0, NSUP, super_body, 0)
    for b in range(2):
      pltpu.make_async_copy(out_bufs[b], out_dst(0), souts[b]).wait()

  return call


def kernel(indices, W):
  idx_t = jnp.swapaxes(indices.astype(jnp.int32), 0, 1).reshape(-1)
  info = plsc.get_sparse_core_info()
  call = _make_sc_call(info.num_cores, info.num_subcores)
  flat = call(idx_t, W)
  return (flat.reshape(L, B // LANES, VOCAB, LANES)
          .transpose(1, 3, 0, 2)
          .reshape(B, L, VOCAB))
